# baseline (speedup 1.0000x reference)
import jax
import jax.numpy as jnp
from jax.experimental import pallas as pl
from jax.experimental.pallas import tpu as pltpu

N = 100000
K = 128
BN = 2000
NB = N // BN
BR = 3 * BN


def _tc_body(x_ref, out_ref):
    x = x_ref[...]
    x1 = pltpu.roll(x, BR - 1, 0)
    x2 = pltpu.roll(x, BR - 2, 0)
    y = x * x1 * x2
    out_ref[...] = jnp.sum(y, axis=-1)[None, None]


@jax.jit
def kernel(triples):
    flat = triples.reshape(3 * N, K)
    out = pl.pallas_call(
        _tc_body,
        grid=(NB,),
        in_specs=[pl.BlockSpec((BR, K), lambda i: (i, 0))],
        out_specs=pl.BlockSpec((1, 1, BR), lambda i: (i, 0, 0)),
        out_shape=jax.ShapeDtypeStruct((NB, 1, BR), jnp.float32),
    )(flat)
    return out.reshape(3 * N)[::3]
